# Initial kernel scaffold; baseline (speedup 1.0000x reference)
import jax
import jax.numpy as jnp
from jax.experimental import pallas as pl


def kernel(x):
    raise NotImplementedError("write your pallas kernel here")


# speedup vs baseline: 1.9233x; 1.9233x over previous
import functools

import jax
import jax.numpy as jnp
from jax import lax
from jax.experimental import pallas as pl
from jax.experimental.pallas import tpu as pltpu
from jax.experimental.pallas import tpu_sc as plsc

B, G, N = 64, 16, 32768
V = N // 16
NW = 32
ROWS_PER_W = B // NW


def _keys_kernel(x_ref, k_ref):
    xb = x_ref[0]
    w = jnp.mean(xb, axis=0, keepdims=True)
    b = lax.bitcast_convert_type(w, jnp.int32)
    kasc = jnp.where(b < 0, jnp.bitwise_not(b),
                     jnp.bitwise_xor(b, jnp.int32(-(2 ** 31))))
    k_ref[0] = jnp.bitwise_not(kasc)


def _make_keys(x):
    out = pl.pallas_call(
        _keys_kernel,
        grid=(B,),
        in_specs=[pl.BlockSpec((1, G, N), lambda i: (i, 0, 0))],
        out_specs=pl.BlockSpec((1, 1, N), lambda i: (i, 0, 0)),
        out_shape=jax.ShapeDtypeStruct((B, 1, N), jnp.int32),
    )(x)
    return out.reshape(B, N)


def _sc_argsort(keys):
    mesh = plsc.VectorSubcoreMesh(core_axis_name="c", subcore_axis_name="s")

    @functools.partial(
        pl.kernel,
        mesh=mesh,
        out_type=jax.ShapeDtypeStruct((B, N), jnp.int32),
        compiler_params=pltpu.CompilerParams(needs_layout_passes=False),
        scratch_types=[
            pltpu.VMEM((N,), jnp.int32),
            pltpu.VMEM((N,), jnp.int32),
            pltpu.VMEM((N,), jnp.int32),
            pltpu.VMEM((4096,), jnp.int32),
        ],
    )
    def body(keys_hbm, out_hbm, keys_v, idx_a, idx_b, hist):
        cid = lax.axis_index("c")
        sid = lax.axis_index("s")
        wid = sid * 2 + cid
        lane = lax.broadcasted_iota(jnp.int32, (16,), 0)
        ones = jnp.full((16,), 1, jnp.int32)

        for rr in range(ROWS_PER_W):
            r = wid * ROWS_PER_W + rr
            pltpu.sync_copy(keys_hbm.at[r], keys_v)

            def init_body(v, c):
                idx_a[pl.ds(v * 16, 16)] = lane * V + v
                return c
            lax.fori_loop(0, V, init_body, 0)

            for p in range(4):
                src, dst = (idx_a, idx_b) if p % 2 == 0 else (idx_b, idx_a)
                shv = jnp.full((16,), 8 * p, jnp.int32)

                def zero_body(i, c):
                    hist[pl.ds(i * 16, 16)] = jnp.zeros((16,), jnp.int32)
                    return c
                lax.fori_loop(0, 256, zero_body, 0)

                def hist_body(v, c, src=src, shv=shv, p=p):
                    idx = src[pl.ds(v * 16, 16)]
                    kk = plsc.load_gather(keys_v, [idx])
                    if p == 0:
                        d = jnp.bitwise_and(kk, jnp.int32(255))
                    else:
                        d = jnp.bitwise_and(
                            lax.shift_right_logical(kk, shv), jnp.int32(255))
                    slot = jnp.bitwise_or(
                        lax.shift_left(d, jnp.full((16,), 4, jnp.int32)), lane)
                    plsc.addupdate_scatter(hist, [slot], ones)
                    return c
                lax.fori_loop(0, V, hist_body, 0)

                def scan_body(i, carry):
                    h = hist[pl.ds(i * 16, 16)]
                    inc = plsc.cumsum(h)
                    hist[pl.ds(i * 16, 16)] = inc - h + carry
                    return carry + jnp.sum(h)
                lax.fori_loop(0, 256, scan_body, jnp.int32(0))

                def perm_body(v, c, src=src, dst=dst, shv=shv, p=p):
                    idx = src[pl.ds(v * 16, 16)]
                    kk = plsc.load_gather(keys_v, [idx])
                    if p == 0:
                        d = jnp.bitwise_and(kk, jnp.int32(255))
                    else:
                        d = jnp.bitwise_and(
                            lax.shift_right_logical(kk, shv), jnp.int32(255))
                    slot = jnp.bitwise_or(
                        lax.shift_left(d, jnp.full((16,), 4, jnp.int32)), lane)
                    q = plsc.load_gather(hist, [slot])
                    plsc.store_scatter(hist, [slot], q + ones)
                    if p == 3:
                        a = q
                    else:
                        a = jnp.bitwise_or(
                            lax.shift_left(jnp.bitwise_and(q, jnp.int32(V - 1)),
                                           jnp.full((16,), 4, jnp.int32)),
                            lax.shift_right_logical(
                                q, jnp.full((16,), 11, jnp.int32)))
                    plsc.store_scatter(dst, [a], idx)
                    return c
                lax.fori_loop(0, V, perm_body, 0)

            pltpu.sync_copy(idx_a, out_hbm.at[r])

    return body(keys)


def kernel(x):
    return _sc_argsort(_make_keys(x))


# speedup vs baseline: 2.4464x; 1.2719x over previous
import functools

import jax
import jax.numpy as jnp
from jax import lax
from jax.experimental import pallas as pl
from jax.experimental.pallas import tpu as pltpu
from jax.experimental.pallas import tpu_sc as plsc

B, G, N = 64, 16, 32768
V = N // 16
NW = 32
ROWS_PER_W = B // NW
U = 4


def _keys_kernel(x_ref, k_ref):
    xb = x_ref[0]
    w = jnp.mean(xb, axis=0, keepdims=True)
    b = lax.bitcast_convert_type(w, jnp.int32)
    kasc = jnp.where(b < 0, jnp.bitwise_not(b),
                     jnp.bitwise_xor(b, jnp.int32(-(2 ** 31))))
    k_ref[0] = jnp.bitwise_not(kasc)


def _make_keys(x):
    out = pl.pallas_call(
        _keys_kernel,
        grid=(B,),
        in_specs=[pl.BlockSpec((1, G, N), lambda i: (i, 0, 0))],
        out_specs=pl.BlockSpec((1, 1, N), lambda i: (i, 0, 0)),
        out_shape=jax.ShapeDtypeStruct((B, 1, N), jnp.int32),
    )(x)
    return out.reshape(B, N)


def _sc_argsort(keys_s):
    mesh = plsc.VectorSubcoreMesh(core_axis_name="c", subcore_axis_name="s")

    @functools.partial(
        pl.kernel,
        mesh=mesh,
        out_type=jax.ShapeDtypeStruct((B, N), jnp.int32),
        compiler_params=pltpu.CompilerParams(needs_layout_passes=False),
        scratch_types=[
            pltpu.VMEM((N,), jnp.int32),
            pltpu.VMEM((N,), jnp.int32),
            pltpu.VMEM((N,), jnp.int32),
            pltpu.VMEM((4096,), jnp.int32),
            pltpu.VMEM((4096,), jnp.int32),
        ],
    )
    def body(keys_hbm, out_hbm, keys_v, pay_a, pay_b, hist_a, hist_b):
        cid = lax.axis_index("c")
        sid = lax.axis_index("s")
        wid = sid * 2 + cid
        lane = lax.broadcasted_iota(jnp.int32, (16,), 0)
        ones = jnp.full((16,), 1, jnp.int32)
        zeros = jnp.zeros((16,), jnp.int32)

        def dig(k, sh):
            if sh:
                k = k >> sh
            return jnp.bitwise_and(k, jnp.int32(255))

        def slot_of(d):
            return jnp.bitwise_or(d << 4, lane)

        def fetch_add(h, slot):
            q = plsc.load_gather(h, [slot])
            plsc.store_scatter(h, [slot], q + ones)
            return q

        def remap(q):
            return jnp.bitwise_or((jnp.bitwise_and(q, jnp.int32(V - 1))) << 4,
                                  q >> 11)

        def hist_next(h, d, q):
            snext = jnp.bitwise_or(d << 4, q >> 11)
            cnt, last = plsc.scan_count(snext)
            plsc.addupdate_scatter(h, [snext], cnt, mask=last)

        def zero_init(i, c):
            for u in range(U):
                hist_a[pl.ds(i * 16 * U + u * 16, 16)] = zeros
            return c
        lax.fori_loop(0, 256 // U, zero_init, 0)

        def make_scan(h_scan, h_zero):
            def scan_body(i, carry):
                hh = h_scan[pl.ds(i * 16, 16)]
                inc = plsc.cumsum(hh)
                h_scan[pl.ds(i * 16, 16)] = inc - hh + carry
                h_zero[pl.ds(i * 16, 16)] = zeros
                return carry + inc[15]
            return scan_body

        for rr in range(ROWS_PER_W):
            r = wid * ROWS_PER_W + rr
            pltpu.sync_copy(keys_hbm.at[r], keys_v)

            def sw0(i, c):
                for u in range(U):
                    k = keys_v[pl.ds(i * 16 * U + u * 16, 16)]
                    plsc.addupdate_scatter(hist_a, [slot_of(dig(k, 0))], ones)
                return c
            lax.fori_loop(0, V // U, sw0, 0)

            lax.fori_loop(0, 256, make_scan(hist_a, hist_b), jnp.int32(0))

            def p0(i, c):
                for u in range(U):
                    a0 = i * 16 * U + u * 16
                    k = keys_v[pl.ds(a0, 16)]
                    payload = lane + a0
                    q = fetch_add(hist_a, slot_of(dig(k, 0)))
                    plsc.store_scatter(pay_a, [remap(q)], payload)
                    hist_next(hist_b, dig(k, 8), q)
                return c
            lax.fori_loop(0, V // U, p0, 0)

            lax.fori_loop(0, 256, make_scan(hist_b, hist_a), jnp.int32(0))

            def p1(i, c):
                for u in range(U):
                    a0 = i * 16 * U + u * 16
                    pay = pay_a[pl.ds(a0, 16)]
                    k = plsc.load_gather(keys_v, [pay])
                    q = fetch_add(hist_b, slot_of(dig(k, 8)))
                    plsc.store_scatter(pay_b, [remap(q)], pay)
                    hist_next(hist_a, dig(k, 16), q)
                return c
            lax.fori_loop(0, V // U, p1, 0)

            lax.fori_loop(0, 256, make_scan(hist_a, hist_b), jnp.int32(0))

            def p2(i, c):
                for u in range(U):
                    a0 = i * 16 * U + u * 16
                    pay = pay_b[pl.ds(a0, 16)]
                    k = plsc.load_gather(keys_v, [pay])
                    q = fetch_add(hist_a, slot_of(dig(k, 16)))
                    plsc.store_scatter(pay_a, [remap(q)], pay)
                    hist_next(hist_b, dig(k, 24), q)
                return c
            lax.fori_loop(0, V // U, p2, 0)

            lax.fori_loop(0, 256, make_scan(hist_b, hist_a), jnp.int32(0))

            def p3(i, c):
                for u in range(U):
                    a0 = i * 16 * U + u * 16
                    pay = pay_a[pl.ds(a0, 16)]
                    k = plsc.load_gather(keys_v, [pay])
                    q = fetch_add(hist_b, slot_of(dig(k, 24)))
                    orig = jnp.bitwise_or(
                        (jnp.bitwise_and(pay, jnp.int32(15))) << 11, pay >> 4)
                    plsc.store_scatter(pay_b, [q], orig)
                return c
            lax.fori_loop(0, V // U, p3, 0)

            if rr + 1 < ROWS_PER_W:
                def zb(i, c):
                    for u in range(U):
                        hist_b[pl.ds(i * 16 * U + u * 16, 16)] = zeros
                    return c
                lax.fori_loop(0, 256 // U, zb, 0)

            pltpu.sync_copy(pay_b, out_hbm.at[r])

    return body(keys_s)


def kernel(x):
    keys = _make_keys(x)
    keys_s = keys.reshape(B, 16, V).swapaxes(1, 2).reshape(B, N)
    return _sc_argsort(keys_s)
